# Initial kernel scaffold; baseline (speedup 1.0000x reference)
import jax
import jax.numpy as jnp
from jax.experimental import pallas as pl


def kernel(x, edge_index, W1, b1, W2, b2):
    raise NotImplementedError("write your pallas kernel here")


# speedup vs baseline: 13.3539x; 13.3539x over previous
import functools

import jax
import jax.numpy as jnp
from jax import lax
from jax.experimental import pallas as pl
from jax.experimental.pallas import tpu as pltpu
from jax.experimental.pallas import tpu_sc as plsc

_N = 10000
_E = 320000
_NC = 2
_NS = 16
_BLK = 128

_ROWS = 624
_ROWS_LAST = _N - (_NS - 1) * _ROWS

_SEG_CHUNK = 157 * _BLK
_SEG_NBLK_LAST = 145

_HALF_STRIDE = 78 * _BLK

_MESH = dict(core_axis_name="c", subcore_axis_name="s", num_cores=_NC,
             num_subcores=_NS)


def _zero_acc(zeros, acc, s):
  @pl.when(s < _NS - 1)
  def _():
    r0 = pl.multiple_of(s * _ROWS, 8)
    pltpu.sync_copy(zeros.at[pl.ds(r0, _ROWS)], acc.at[pl.ds(r0, _ROWS)])

  @pl.when(s == _NS - 1)
  def _():
    r0 = (_NS - 1) * _ROWS
    pltpu.sync_copy(zeros.at[pl.ds(r0, _ROWS_LAST)],
                    acc.at[pl.ds(r0, _ROWS_LAST)])


def _write_acc(acc, out, s, out_base):
  @pl.when(s < _NS - 1)
  def _():
    r0 = pl.multiple_of(s * _ROWS, 8)
    pltpu.sync_copy(acc.at[pl.ds(r0, _ROWS)],
                    out.at[pl.ds(pl.multiple_of(out_base + r0, 8), _ROWS)])

  @pl.when(s == _NS - 1)
  def _():
    r0 = (_NS - 1) * _ROWS
    pltpu.sync_copy(
        acc.at[pl.ds(r0, _ROWS_LAST)],
        out.at[pl.ds(pl.multiple_of(out_base + r0, 8), _ROWS_LAST)])


@functools.partial(
    pl.kernel,
    out_type=jax.ShapeDtypeStruct((_NC * _N, 128), jnp.float32),
    mesh=plsc.VectorSubcoreMesh(**_MESH),
    scratch_types=[
        pltpu.VMEM_SHARED((_N, 128), jnp.float32),
        pltpu.VMEM((_BLK,), jnp.int32),
        pltpu.VMEM((_BLK,), jnp.int32),
        pltpu.VMEM((_BLK, 128), jnp.float32),
        pltpu.SemaphoreType.DMA,
    ],
)
def _seg1(table, src2, dst, zeros, out, acc, idxs, idxd, rows, sem):
  c = lax.axis_index("c")
  s = lax.axis_index("s")
  _zero_acc(zeros, acc, s)
  plsc.subcore_barrier()

  ebase = s * _SEG_CHUNK
  nblk = jnp.where(s == _NS - 1, _SEG_NBLK_LAST, _SEG_CHUNK // _BLK)

  def body(g, carry):
    off = ebase + g * _BLK
    pltpu.sync_copy(src2.at[pl.ds(c * _E + off, _BLK)], idxs)
    pltpu.sync_copy(dst.at[pl.ds(off, _BLK)], idxd)
    pltpu.async_copy(table.at[idxs], rows, sem).wait()
    pltpu.sync_copy(rows, acc.at[idxd], add=True)
    return carry

  lax.fori_loop(0, nblk, body, 0)
  plsc.subcore_barrier()
  _write_acc(acc, out, s, c * _N)


@functools.partial(
    pl.kernel,
    out_type=jax.ShapeDtypeStruct((_NC * _N, 128), jnp.float32),
    mesh=plsc.VectorSubcoreMesh(**_MESH),
    scratch_types=[
        pltpu.VMEM_SHARED((_N, 128), jnp.float32),
        pltpu.VMEM((_BLK,), jnp.int32),
        pltpu.VMEM((_BLK,), jnp.int32),
        pltpu.VMEM((_BLK, 128), jnp.float32),
        pltpu.SemaphoreType.DMA,
    ],
)
def _seg2(table, src, dst, zeros, out, acc, idxs, idxd, rows, sem):
  c = lax.axis_index("c")
  s = lax.axis_index("s")
  _zero_acc(zeros, acc, s)
  plsc.subcore_barrier()

  base = c * (_E // _NC) + s * _HALF_STRIDE + jnp.minimum(s, 2) * _BLK
  nblk = jnp.where(s < 2, 79, 78)

  def body(g, carry):
    off = base + g * _BLK
    pltpu.sync_copy(src.at[pl.ds(off, _BLK)], idxs)
    pltpu.sync_copy(dst.at[pl.ds(off, _BLK)], idxd)
    pltpu.async_copy(table.at[idxs], rows, sem).wait()
    pltpu.sync_copy(rows, acc.at[idxd], add=True)
    return carry

  lax.fori_loop(0, nblk, body, 0)
  plsc.subcore_barrier()
  _write_acc(acc, out, s, c * _N)


@functools.partial(
    pl.kernel,
    out_type=jax.ShapeDtypeStruct((_NC * _N,), jnp.float32),
    mesh=plsc.VectorSubcoreMesh(**_MESH),
    scratch_types=[
        pltpu.VMEM_SHARED((_N,), jnp.float32),
        pltpu.VMEM((_BLK,), jnp.int32),
        pltpu.VMEM((_BLK,), jnp.float32),
        pltpu.VMEM((_N,), jnp.float32),
    ],
)
def _deg_kernel(dst, zeros1, out, deg, idxd, ones, vbuf):
  c = lax.axis_index("c")
  s = lax.axis_index("s")

  @pl.when(s == 0)
  def _zero():
    pltpu.sync_copy(zeros1, deg)

  for j in range(_BLK // 16):
    ones[pl.ds(16 * j, 16)] = jnp.full((16,), 1.0, jnp.float32)
  plsc.subcore_barrier()

  base = c * (_E // _NC) + s * _HALF_STRIDE + jnp.minimum(s, 2) * _BLK
  nblk = jnp.where(s < 2, 79, 78)

  def body(g, carry):
    off = base + g * _BLK
    pltpu.sync_copy(dst.at[pl.ds(off, _BLK)], idxd)
    pltpu.sync_copy(ones, deg.at[idxd], add=True)
    return carry

  lax.fori_loop(0, nblk, body, 0)
  plsc.subcore_barrier()

  @pl.when(s == 0)
  def _writeout():
    pltpu.sync_copy(deg, vbuf)
    pltpu.sync_copy(vbuf, out.at[pl.ds(pl.multiple_of(c * _N, 8), _N)])


_BN = 1000


def _dinv_block(degp_ref):
  deg = degp_ref[0, 0, :] + degp_ref[0, 1, :] + 1.0
  return lax.rsqrt(deg)


def _mm1_body(x_ref, w1_ref, degp_ref, y_ref):
  dinv = _dinv_block(degp_ref)
  y = jnp.dot(x_ref[...], w1_ref[...], preferred_element_type=jnp.float32)
  y = y * dinv[:, None]
  y_ref[0] = y[:, :128]
  y_ref[1] = y[:, 128:]


_mm1 = pl.pallas_call(
    _mm1_body,
    grid=(_N // _BN,),
    in_specs=[
        pl.BlockSpec((_BN, 128), lambda i: (i, 0)),
        pl.BlockSpec((128, 256), lambda i: (0, 0)),
        pl.BlockSpec((1, 2, _BN), lambda i: (i, 0, 0)),
    ],
    out_specs=pl.BlockSpec((2, _BN, 128), lambda i: (0, i, 0)),
    out_shape=jax.ShapeDtypeStruct((2, _N, 128), jnp.float32),
)


def _mm2_body(acc_ref, y1_ref, degp_ref, b1_ref, w2_ref, y2_ref):
  dinv = _dinv_block(degp_ref)
  pre = jnp.concatenate(
      [acc_ref[0] + y1_ref[0], acc_ref[1] + y1_ref[1]], axis=1)
  h = jnp.maximum(pre * dinv[:, None] + b1_ref[0, :], 0.0)
  y2 = jnp.dot(h, w2_ref[...], preferred_element_type=jnp.float32)
  y2_ref[...] = y2 * dinv[:, None]


_mm2 = pl.pallas_call(
    _mm2_body,
    grid=(_N // _BN,),
    in_specs=[
        pl.BlockSpec((2, _BN, 128), lambda i: (0, i, 0)),
        pl.BlockSpec((2, _BN, 128), lambda i: (0, i, 0)),
        pl.BlockSpec((1, 2, _BN), lambda i: (i, 0, 0)),
        pl.BlockSpec((1, 256), lambda i: (0, 0)),
        pl.BlockSpec((256, 128), lambda i: (0, 0)),
    ],
    out_specs=pl.BlockSpec((_BN, 128), lambda i: (i, 0)),
    out_shape=jax.ShapeDtypeStruct((_N, 128), jnp.float32),
)


def _fin_body(accp_ref, y2_ref, degp_ref, b2_ref, o_ref):
  dinv = _dinv_block(degp_ref)
  acc = accp_ref[0] + accp_ref[1]
  o_ref[...] = (acc + y2_ref[...]) * dinv[:, None] + b2_ref[0, :]


_fin = pl.pallas_call(
    _fin_body,
    grid=(_N // _BN,),
    in_specs=[
        pl.BlockSpec((2, _BN, 128), lambda i: (0, i, 0)),
        pl.BlockSpec((_BN, 128), lambda i: (i, 0)),
        pl.BlockSpec((1, 2, _BN), lambda i: (i, 0, 0)),
        pl.BlockSpec((1, 128), lambda i: (0, 0)),
    ],
    out_specs=pl.BlockSpec((_BN, 128), lambda i: (i, 0)),
    out_shape=jax.ShapeDtypeStruct((_N, 128), jnp.float32),
)


def kernel(x, edge_index, W1, b1, W2, b2):
  src, dst = edge_index[0], edge_index[1]
  src2 = jnp.concatenate([src, src + _N])
  zeros1 = jnp.zeros((_N,), jnp.float32)
  zeros128 = jnp.zeros((_N, 128), jnp.float32)

  degp = _deg_kernel(dst, zeros1).reshape(2, _N)
  degp3 = degp.reshape(2, _N // _BN, _BN).transpose(1, 0, 2)
  y1 = _mm1(x, W1, degp3)
  acc1 = _seg1(y1.reshape(_NC * _N, 128), src2, dst, zeros128)
  y2 = _mm2(acc1.reshape(2, _N, 128), y1, degp3, b1.reshape(1, 256), W2)
  acc2p = _seg2(y2, src, dst, zeros128)
  return _fin(acc2p.reshape(2, _N, 128), y2, degp3, b2.reshape(1, 128))


# speedup vs baseline: 18.6863x; 1.3993x over previous
import functools

import jax
import jax.numpy as jnp
from jax import lax
from jax.experimental import pallas as pl
from jax.experimental.pallas import tpu as pltpu
from jax.experimental.pallas import tpu_sc as plsc

_N = 10000
_E = 320000
_NC = 2
_NS = 16
_BLK = 128

_ROWS = 624
_ROWS_LAST = _N - (_NS - 1) * _ROWS

_SEG_CHUNK = 156 * _BLK

_HALF_STRIDE = 78 * _BLK

_MESH = dict(core_axis_name="c", subcore_axis_name="s", num_cores=_NC,
             num_subcores=_NS)


def _zero_acc(zeros, acc, s):
  @pl.when(s < _NS - 1)
  def _():
    r0 = pl.multiple_of(s * _ROWS, 8)
    pltpu.sync_copy(zeros.at[pl.ds(r0, _ROWS)], acc.at[pl.ds(r0, _ROWS)])

  @pl.when(s == _NS - 1)
  def _():
    r0 = (_NS - 1) * _ROWS
    pltpu.sync_copy(zeros.at[pl.ds(r0, _ROWS_LAST)],
                    acc.at[pl.ds(r0, _ROWS_LAST)])


def _write_acc(acc, out, s, out_base):
  @pl.when(s < _NS - 1)
  def _():
    r0 = pl.multiple_of(s * _ROWS, 8)
    pltpu.sync_copy(acc.at[pl.ds(r0, _ROWS)],
                    out.at[pl.ds(pl.multiple_of(out_base + r0, 8), _ROWS)])

  @pl.when(s == _NS - 1)
  def _():
    r0 = (_NS - 1) * _ROWS
    pltpu.sync_copy(
        acc.at[pl.ds(r0, _ROWS_LAST)],
        out.at[pl.ds(pl.multiple_of(out_base + r0, 8), _ROWS_LAST)])


_SEG_SCRATCH = [
    pltpu.VMEM_SHARED((_N, 128), jnp.float32),
    pltpu.VMEM((_BLK,), jnp.int32),
    pltpu.VMEM((_BLK,), jnp.int32),
    pltpu.VMEM((_BLK,), jnp.int32),
    pltpu.VMEM((_BLK,), jnp.int32),
    pltpu.VMEM((_BLK, 128), jnp.float32),
    pltpu.VMEM((_BLK, 128), jnp.float32),
    pltpu.SemaphoreType.DMA,
    pltpu.SemaphoreType.DMA,
    pltpu.SemaphoreType.DMA,
    pltpu.SemaphoreType.DMA,
    pltpu.SemaphoreType.DMA,
]


def _seg_pipeline(table, srcarr, dstarr, acc, idxs0, idxs1, idxd0, idxd1,
                  rows0, rows1, semi, semg0, semg1, sems0, sems1,
                  soff, doff, npairs):
  def body(h, carry):
    o = h * (2 * _BLK)

    @pl.when(h >= 1)
    def _drain_prev():
      pltpu.make_async_copy(rows0, acc.at[idxd0], sems0).wait()
      pltpu.make_async_copy(rows1, acc.at[idxd1], sems1).wait()

    d1 = pltpu.async_copy(srcarr.at[pl.ds(soff + o, _BLK)], idxs0, semi)
    d2 = pltpu.async_copy(srcarr.at[pl.ds(soff + o + _BLK, _BLK)], idxs1,
                          semi)
    d3 = pltpu.async_copy(dstarr.at[pl.ds(doff + o, _BLK)], idxd0, semi)
    d4 = pltpu.async_copy(dstarr.at[pl.ds(doff + o + _BLK, _BLK)], idxd1,
                          semi)
    d1.wait()
    d2.wait()
    d3.wait()
    d4.wait()
    g0 = pltpu.async_copy(table.at[idxs0], rows0, semg0)
    g1 = pltpu.async_copy(table.at[idxs1], rows1, semg1)
    g0.wait()
    pltpu.async_copy(rows0, acc.at[idxd0], sems0, add=True)
    g1.wait()
    pltpu.async_copy(rows1, acc.at[idxd1], sems1, add=True)
    return carry

  lax.fori_loop(0, npairs, body, 0)
  pltpu.make_async_copy(rows0, acc.at[idxd0], sems0).wait()
  pltpu.make_async_copy(rows1, acc.at[idxd1], sems1).wait()


@functools.partial(
    pl.kernel,
    out_type=jax.ShapeDtypeStruct((_NC * _N, 128), jnp.float32),
    mesh=plsc.VectorSubcoreMesh(**_MESH),
    scratch_types=list(_SEG_SCRATCH),
)
def _seg1(table, src2, dst, zeros, out, acc, idxs0, idxs1, idxd0, idxd1,
          rows0, rows1, semi, semg0, semg1, sems0, sems1):
  c = lax.axis_index("c")
  s = lax.axis_index("s")
  _zero_acc(zeros, acc, s)
  plsc.subcore_barrier()

  ebase = s * _SEG_CHUNK
  npairs = jnp.where(s == _NS - 1, 80, _SEG_CHUNK // (2 * _BLK))
  _seg_pipeline(table, src2, dst, acc, idxs0, idxs1, idxd0, idxd1,
                rows0, rows1, semi, semg0, semg1, sems0, sems1,
                c * _E + ebase, ebase, npairs)
  plsc.subcore_barrier()
  _write_acc(acc, out, s, c * _N)


@functools.partial(
    pl.kernel,
    out_type=jax.ShapeDtypeStruct((_NC * _N, 128), jnp.float32),
    mesh=plsc.VectorSubcoreMesh(**_MESH),
    scratch_types=list(_SEG_SCRATCH),
)
def _seg2(table, src, dst, zeros, out, acc, idxs0, idxs1, idxd0, idxd1,
          rows0, rows1, semi, semg0, semg1, sems0, sems1):
  c = lax.axis_index("c")
  s = lax.axis_index("s")
  _zero_acc(zeros, acc, s)
  plsc.subcore_barrier()

  base = c * (_E // _NC) + s * _HALF_STRIDE
  npairs = jnp.where(s == _NS - 1, 40, 39)
  _seg_pipeline(table, src, dst, acc, idxs0, idxs1, idxd0, idxd1,
                rows0, rows1, semi, semg0, semg1, sems0, sems1,
                base, base, npairs)
  plsc.subcore_barrier()
  _write_acc(acc, out, s, c * _N)


@functools.partial(
    pl.kernel,
    out_type=jax.ShapeDtypeStruct((_NC * _N,), jnp.float32),
    mesh=plsc.VectorSubcoreMesh(**_MESH),
    scratch_types=[
        pltpu.VMEM_SHARED((_N,), jnp.float32),
        pltpu.VMEM((_BLK,), jnp.int32),
        pltpu.VMEM((_BLK,), jnp.float32),
        pltpu.VMEM((_N,), jnp.float32),
    ],
)
def _deg_kernel(dst, zeros1, out, deg, idxd, ones, vbuf):
  c = lax.axis_index("c")
  s = lax.axis_index("s")

  @pl.when(s == 0)
  def _zero():
    pltpu.sync_copy(zeros1, deg)

  for j in range(_BLK // 16):
    ones[pl.ds(16 * j, 16)] = jnp.full((16,), 1.0, jnp.float32)
  plsc.subcore_barrier()

  base = c * (_E // _NC) + s * _HALF_STRIDE + jnp.minimum(s, 2) * _BLK
  nblk = jnp.where(s < 2, 79, 78)

  def body(g, carry):
    off = base + g * _BLK
    pltpu.sync_copy(dst.at[pl.ds(off, _BLK)], idxd)
    pltpu.sync_copy(ones, deg.at[idxd], add=True)
    return carry

  lax.fori_loop(0, nblk, body, 0)
  plsc.subcore_barrier()

  @pl.when(s == 0)
  def _writeout():
    pltpu.sync_copy(deg, vbuf)
    pltpu.sync_copy(vbuf, out.at[pl.ds(pl.multiple_of(c * _N, 8), _N)])


_BN = 1000


def _dinv_block(degp_ref):
  deg = degp_ref[0, 0, :] + degp_ref[0, 1, :] + 1.0
  return lax.rsqrt(deg)


def _mm1_body(x_ref, w1_ref, degp_ref, y_ref):
  dinv = _dinv_block(degp_ref)
  y = jnp.dot(x_ref[...], w1_ref[...], preferred_element_type=jnp.float32)
  y = y * dinv[:, None]
  y_ref[0] = y[:, :128]
  y_ref[1] = y[:, 128:]


_mm1 = pl.pallas_call(
    _mm1_body,
    grid=(_N // _BN,),
    in_specs=[
        pl.BlockSpec((_BN, 128), lambda i: (i, 0)),
        pl.BlockSpec((128, 256), lambda i: (0, 0)),
        pl.BlockSpec((1, 2, _BN), lambda i: (i, 0, 0)),
    ],
    out_specs=pl.BlockSpec((2, _BN, 128), lambda i: (0, i, 0)),
    out_shape=jax.ShapeDtypeStruct((2, _N, 128), jnp.float32),
)


def _mm2_body(acc_ref, y1_ref, degp_ref, b1_ref, w2_ref, y2_ref):
  dinv = _dinv_block(degp_ref)
  pre = jnp.concatenate(
      [acc_ref[0] + y1_ref[0], acc_ref[1] + y1_ref[1]], axis=1)
  h = jnp.maximum(pre * dinv[:, None] + b1_ref[0, :], 0.0)
  y2 = jnp.dot(h, w2_ref[...], preferred_element_type=jnp.float32)
  y2_ref[...] = y2 * dinv[:, None]


_mm2 = pl.pallas_call(
    _mm2_body,
    grid=(_N // _BN,),
    in_specs=[
        pl.BlockSpec((2, _BN, 128), lambda i: (0, i, 0)),
        pl.BlockSpec((2, _BN, 128), lambda i: (0, i, 0)),
        pl.BlockSpec((1, 2, _BN), lambda i: (i, 0, 0)),
        pl.BlockSpec((1, 256), lambda i: (0, 0)),
        pl.BlockSpec((256, 128), lambda i: (0, 0)),
    ],
    out_specs=pl.BlockSpec((_BN, 128), lambda i: (i, 0)),
    out_shape=jax.ShapeDtypeStruct((_N, 128), jnp.float32),
)


def _fin_body(accp_ref, y2_ref, degp_ref, b2_ref, o_ref):
  dinv = _dinv_block(degp_ref)
  acc = accp_ref[0] + accp_ref[1]
  o_ref[...] = (acc + y2_ref[...]) * dinv[:, None] + b2_ref[0, :]


_fin = pl.pallas_call(
    _fin_body,
    grid=(_N // _BN,),
    in_specs=[
        pl.BlockSpec((2, _BN, 128), lambda i: (0, i, 0)),
        pl.BlockSpec((_BN, 128), lambda i: (i, 0)),
        pl.BlockSpec((1, 2, _BN), lambda i: (i, 0, 0)),
        pl.BlockSpec((1, 128), lambda i: (0, 0)),
    ],
    out_specs=pl.BlockSpec((_BN, 128), lambda i: (i, 0)),
    out_shape=jax.ShapeDtypeStruct((_N, 128), jnp.float32),
)


def kernel(x, edge_index, W1, b1, W2, b2):
  src, dst = edge_index[0], edge_index[1]
  src2 = jnp.concatenate([src, src + _N])
  zeros1 = jnp.zeros((_N,), jnp.float32)
  zeros128 = jnp.zeros((_N, 128), jnp.float32)

  degp = _deg_kernel(dst, zeros1).reshape(2, _N)
  degp3 = degp.reshape(2, _N // _BN, _BN).transpose(1, 0, 2)
  y1 = _mm1(x, W1, degp3)
  acc1 = _seg1(y1.reshape(_NC * _N, 128), src2, dst, zeros128)
  y2 = _mm2(acc1.reshape(2, _N, 128), y1, degp3, b1.reshape(1, 256), W2)
  acc2p = _seg2(y2, src, dst, zeros128)
  return _fin(acc2p.reshape(2, _N, 128), y2, degp3, b2.reshape(1, 128))


# speedup vs baseline: 19.7034x; 1.0544x over previous
import functools

import jax
import jax.numpy as jnp
from jax import lax
from jax.experimental import pallas as pl
from jax.experimental.pallas import tpu as pltpu
from jax.experimental.pallas import tpu_sc as plsc

_N = 10000
_E = 320000
_NC = 2
_NS = 16
_BLK = 128

_ROWS = 624
_ROWS_LAST = _N - (_NS - 1) * _ROWS

_HALF_STRIDE = 78 * _BLK

_MESH = dict(core_axis_name="c", subcore_axis_name="s", num_cores=_NC,
             num_subcores=_NS)


def _zero_acc(zeros, acc, s):
  @pl.when(s < _NS - 1)
  def _():
    r0 = pl.multiple_of(s * _ROWS, 8)
    pltpu.sync_copy(zeros.at[pl.ds(r0, _ROWS)], acc.at[pl.ds(r0, _ROWS)])

  @pl.when(s == _NS - 1)
  def _():
    r0 = (_NS - 1) * _ROWS
    pltpu.sync_copy(zeros.at[pl.ds(r0, _ROWS_LAST)],
                    acc.at[pl.ds(r0, _ROWS_LAST)])


def _write_acc(acc, out, s, out_base):
  @pl.when(s < _NS - 1)
  def _():
    r0 = pl.multiple_of(s * _ROWS, 8)
    pltpu.sync_copy(acc.at[pl.ds(r0, _ROWS)],
                    out.at[pl.ds(pl.multiple_of(out_base + r0, 8), _ROWS)])

  @pl.when(s == _NS - 1)
  def _():
    r0 = (_NS - 1) * _ROWS
    pltpu.sync_copy(
        acc.at[pl.ds(r0, _ROWS_LAST)],
        out.at[pl.ds(pl.multiple_of(out_base + r0, 8), _ROWS_LAST)])


_Q = 3

_SEG_SCRATCH = (
    [pltpu.VMEM_SHARED((_N, 128), jnp.float32)]
    + [pltpu.VMEM((_BLK,), jnp.int32) for _ in range(2 * _Q)]
    + [pltpu.VMEM((_BLK, 128), jnp.float32) for _ in range(_Q)]
    + [pltpu.SemaphoreType.DMA for _ in range(3)]
)


def _seg_pipeline(table, srcarr, dstarr, acc, idxs, idxd, rows,
                  semi, semg, sems, soff, doff, nquads):
  def body(h, carry):
    o = h * (_Q * _BLK)

    @pl.when(h >= 1)
    def _drain_prev():
      for j in range(_Q):
        pltpu.make_async_copy(rows[j], acc.at[idxd[j]], sems).wait()

    di = []
    for j in range(_Q):
      di.append(pltpu.async_copy(
          srcarr.at[pl.ds(soff + o + j * _BLK, _BLK)], idxs[j], semi))
      di.append(pltpu.async_copy(
          dstarr.at[pl.ds(doff + o + j * _BLK, _BLK)], idxd[j], semi))
    for d in di:
      d.wait()
    dg = [pltpu.async_copy(table.at[idxs[j]], rows[j], semg)
          for j in range(_Q)]
    for d in dg:
      d.wait()
    for j in range(_Q):
      pltpu.async_copy(rows[j], acc.at[idxd[j]], sems, add=True)
    return carry

  lax.fori_loop(0, nquads, body, 0)
  for j in range(_Q):
    pltpu.make_async_copy(rows[j], acc.at[idxd[j]], sems).wait()


def _seg_tail(table, srcarr, dstarr, acc, idxs0, idxd0, rows0, semg,
              soff, doff):
  pltpu.sync_copy(srcarr.at[pl.ds(soff, _BLK)], idxs0)
  pltpu.sync_copy(dstarr.at[pl.ds(doff, _BLK)], idxd0)
  pltpu.async_copy(table.at[idxs0], rows0, semg).wait()
  pltpu.sync_copy(rows0, acc.at[idxd0], add=True)


@functools.partial(
    pl.kernel,
    out_type=jax.ShapeDtypeStruct((_NC * _N, 128), jnp.float32),
    mesh=plsc.VectorSubcoreMesh(**_MESH),
    scratch_types=list(_SEG_SCRATCH),
)
def _seg1(table, src2, dst, zeros, out, acc, *bufs):
  c = lax.axis_index("c")
  s = lax.axis_index("s")
  idxs, idxd, rows = bufs[:_Q], bufs[_Q:2 * _Q], bufs[2 * _Q:3 * _Q]
  semi, semg, sems = bufs[3 * _Q:]
  _zero_acc(zeros, acc, s)
  plsc.subcore_barrier()

  ebase = s * (156 * _BLK)
  ngroups = jnp.where(s == _NS - 1, 53, 52)
  _seg_pipeline(table, src2, dst, acc, idxs, idxd, rows, semi, semg, sems,
                c * _E + ebase, ebase, ngroups)

  @pl.when(s == _NS - 1)
  def _tail():
    _seg_tail(table, src2, dst, acc, idxs[0], idxd[0], rows[0], semg,
              c * _E + 2499 * _BLK, 2499 * _BLK)

  plsc.subcore_barrier()
  _write_acc(acc, out, s, c * _N)


@functools.partial(
    pl.kernel,
    out_type=jax.ShapeDtypeStruct((_NC * _N, 128), jnp.float32),
    mesh=plsc.VectorSubcoreMesh(**_MESH),
    scratch_types=list(_SEG_SCRATCH),
)
def _seg2(table, src, dst, zeros, out, acc, *bufs):
  c = lax.axis_index("c")
  s = lax.axis_index("s")
  idxs, idxd, rows = bufs[:_Q], bufs[_Q:2 * _Q], bufs[2 * _Q:3 * _Q]
  semi, semg, sems = bufs[3 * _Q:]
  _zero_acc(zeros, acc, s)
  plsc.subcore_barrier()

  w = c * _NS + s
  base = w * (78 * _BLK)
  ngroups = jnp.where(w == 2 * _NS - 1, 27, 26)
  _seg_pipeline(table, src, dst, acc, idxs, idxd, rows, semi, semg, sems,
                base, base, ngroups)

  @pl.when(w == 2 * _NS - 1)
  def _tail():
    _seg_tail(table, src, dst, acc, idxs[0], idxd[0], rows[0], semg,
              2499 * _BLK, 2499 * _BLK)

  plsc.subcore_barrier()
  _write_acc(acc, out, s, c * _N)


@functools.partial(
    pl.kernel,
    out_type=jax.ShapeDtypeStruct((_NC * _N,), jnp.float32),
    mesh=plsc.VectorSubcoreMesh(**_MESH),
    scratch_types=[
        pltpu.VMEM_SHARED((_N,), jnp.float32),
        pltpu.VMEM((_BLK,), jnp.int32),
        pltpu.VMEM((_BLK,), jnp.float32),
        pltpu.VMEM((_N,), jnp.float32),
    ],
)
def _deg_kernel(dst, zeros1, out, deg, idxd, ones, vbuf):
  c = lax.axis_index("c")
  s = lax.axis_index("s")

  @pl.when(s == 0)
  def _zero():
    pltpu.sync_copy(zeros1, deg)

  for j in range(_BLK // 16):
    ones[pl.ds(16 * j, 16)] = jnp.full((16,), 1.0, jnp.float32)
  plsc.subcore_barrier()

  base = c * (_E // _NC) + s * _HALF_STRIDE + jnp.minimum(s, 2) * _BLK
  nblk = jnp.where(s < 2, 79, 78)

  def body(g, carry):
    off = base + g * _BLK
    pltpu.sync_copy(dst.at[pl.ds(off, _BLK)], idxd)
    pltpu.sync_copy(ones, deg.at[idxd], add=True)
    return carry

  lax.fori_loop(0, nblk, body, 0)
  plsc.subcore_barrier()

  @pl.when(s == 0)
  def _writeout():
    pltpu.sync_copy(deg, vbuf)
    pltpu.sync_copy(vbuf, out.at[pl.ds(pl.multiple_of(c * _N, 8), _N)])


_BN = 1000


def _dinv_block(degp_ref):
  deg = degp_ref[0, 0, :] + degp_ref[0, 1, :] + 1.0
  return lax.rsqrt(deg)


def _mm1_body(x_ref, w1_ref, degp_ref, y_ref):
  dinv = _dinv_block(degp_ref)
  y = jnp.dot(x_ref[...], w1_ref[...], preferred_element_type=jnp.float32)
  y = y * dinv[:, None]
  y_ref[0] = y[:, :128]
  y_ref[1] = y[:, 128:]


_mm1 = pl.pallas_call(
    _mm1_body,
    grid=(_N // _BN,),
    in_specs=[
        pl.BlockSpec((_BN, 128), lambda i: (i, 0)),
        pl.BlockSpec((128, 256), lambda i: (0, 0)),
        pl.BlockSpec((1, 2, _BN), lambda i: (i, 0, 0)),
    ],
    out_specs=pl.BlockSpec((2, _BN, 128), lambda i: (0, i, 0)),
    out_shape=jax.ShapeDtypeStruct((2, _N, 128), jnp.float32),
)


def _mm2_body(acc_ref, y1_ref, degp_ref, b1_ref, w2_ref, y2_ref):
  dinv = _dinv_block(degp_ref)
  pre = jnp.concatenate(
      [acc_ref[0] + y1_ref[0], acc_ref[1] + y1_ref[1]], axis=1)
  h = jnp.maximum(pre * dinv[:, None] + b1_ref[0, :], 0.0)
  y2 = jnp.dot(h, w2_ref[...], preferred_element_type=jnp.float32)
  y2_ref[...] = y2 * dinv[:, None]


_mm2 = pl.pallas_call(
    _mm2_body,
    grid=(_N // _BN,),
    in_specs=[
        pl.BlockSpec((2, _BN, 128), lambda i: (0, i, 0)),
        pl.BlockSpec((2, _BN, 128), lambda i: (0, i, 0)),
        pl.BlockSpec((1, 2, _BN), lambda i: (i, 0, 0)),
        pl.BlockSpec((1, 256), lambda i: (0, 0)),
        pl.BlockSpec((256, 128), lambda i: (0, 0)),
    ],
    out_specs=pl.BlockSpec((_BN, 128), lambda i: (i, 0)),
    out_shape=jax.ShapeDtypeStruct((_N, 128), jnp.float32),
)


def _fin_body(accp_ref, y2_ref, degp_ref, b2_ref, o_ref):
  dinv = _dinv_block(degp_ref)
  acc = accp_ref[0] + accp_ref[1]
  o_ref[...] = (acc + y2_ref[...]) * dinv[:, None] + b2_ref[0, :]


_fin = pl.pallas_call(
    _fin_body,
    grid=(_N // _BN,),
    in_specs=[
        pl.BlockSpec((2, _BN, 128), lambda i: (0, i, 0)),
        pl.BlockSpec((_BN, 128), lambda i: (i, 0)),
        pl.BlockSpec((1, 2, _BN), lambda i: (i, 0, 0)),
        pl.BlockSpec((1, 128), lambda i: (0, 0)),
    ],
    out_specs=pl.BlockSpec((_BN, 128), lambda i: (i, 0)),
    out_shape=jax.ShapeDtypeStruct((_N, 128), jnp.float32),
)


def kernel(x, edge_index, W1, b1, W2, b2):
  src, dst = edge_index[0], edge_index[1]
  src2 = jnp.concatenate([src, src + _N])
  zeros1 = jnp.zeros((_N,), jnp.float32)
  zeros128 = jnp.zeros((_N, 128), jnp.float32)

  degp = _deg_kernel(dst, zeros1).reshape(2, _N)
  degp3 = degp.reshape(2, _N // _BN, _BN).transpose(1, 0, 2)
  y1 = _mm1(x, W1, degp3)
  acc1 = _seg1(y1.reshape(_NC * _N, 128), src2, dst, zeros128)
  y2 = _mm2(acc1.reshape(2, _N, 128), y1, degp3, b1.reshape(1, 256), W2)
  acc2p = _seg2(y2, src, dst, zeros128)
  return _fin(acc2p.reshape(2, _N, 128), y2, degp3, b2.reshape(1, 128))


# speedup vs baseline: 26.0339x; 1.3213x over previous
import functools

import jax
import jax.numpy as jnp
from jax import lax
from jax.experimental import pallas as pl
from jax.experimental.pallas import tpu as pltpu
from jax.experimental.pallas import tpu_sc as plsc

_N = 10000
_E = 320000
_NC = 2
_NS = 16
_BLK = 128

_ROWS = 624
_ROWS_LAST = _N - (_NS - 1) * _ROWS

_HALF_STRIDE = 78 * _BLK

_MESH = dict(core_axis_name="c", subcore_axis_name="s", num_cores=_NC,
             num_subcores=_NS)

_Q = 3


def _zero_acc(zeros, acc, s):
  @pl.when(s < _NS - 1)
  def _():
    r0 = pl.multiple_of(s * _ROWS, 8)
    pltpu.sync_copy(zeros.at[pl.ds(r0, _ROWS)], acc.at[pl.ds(r0, _ROWS)])

  @pl.when(s == _NS - 1)
  def _():
    r0 = (_NS - 1) * _ROWS
    pltpu.sync_copy(zeros.at[pl.ds(r0, _ROWS_LAST)],
                    acc.at[pl.ds(r0, _ROWS_LAST)])


def _write_acc(acc, out, s, out_base):
  @pl.when(s < _NS - 1)
  def _():
    r0 = pl.multiple_of(s * _ROWS, 8)
    pltpu.sync_copy(acc.at[pl.ds(r0, _ROWS)],
                    out.at[pl.ds(pl.multiple_of(out_base + r0, 8), _ROWS)])

  @pl.when(s == _NS - 1)
  def _():
    r0 = (_NS - 1) * _ROWS
    pltpu.sync_copy(
        acc.at[pl.ds(r0, _ROWS_LAST)],
        out.at[pl.ds(pl.multiple_of(out_base + r0, 8), _ROWS_LAST)])


def _seg_pipeline(table, srcarr, dstarr, acc, idxs, idxd, rows,
                  semi, semg, sems, eoff, ngroups):
  def body(h, carry):
    o = eoff + h * (_Q * _BLK)

    @pl.when(h >= 1)
    def _drain_prev():
      for j in range(_Q):
        pltpu.make_async_copy(rows[j], acc.at[idxd[j]], sems).wait()

    di = []
    for j in range(_Q):
      di.append(pltpu.async_copy(
          srcarr.at[pl.ds(o + j * _BLK, _BLK)], idxs[j], semi))
      di.append(pltpu.async_copy(
          dstarr.at[pl.ds(o + j * _BLK, _BLK)], idxd[j], semi))
    for d in di:
      d.wait()
    dg = [pltpu.async_copy(table.at[idxs[j]], rows[j], semg)
          for j in range(_Q)]
    for d in dg:
      d.wait()
    for j in range(_Q):
      pltpu.async_copy(rows[j], acc.at[idxd[j]], sems, add=True)
    return carry

  lax.fori_loop(0, ngroups, body, 0)
  for j in range(_Q):
    pltpu.make_async_copy(rows[j], acc.at[idxd[j]], sems).wait()


def _seg_tail(table, srcarr, dstarr, acc, idxs0, idxd0, rows0, semg, eoff):
  pltpu.sync_copy(srcarr.at[pl.ds(eoff, _BLK)], idxs0)
  pltpu.sync_copy(dstarr.at[pl.ds(eoff, _BLK)], idxd0)
  pltpu.async_copy(table.at[idxs0], rows0, semg).wait()
  pltpu.sync_copy(rows0, acc.at[idxd0], add=True)


@functools.partial(
    pl.kernel,
    out_type=jax.ShapeDtypeStruct((_NC * _N, 128), jnp.float32),
    mesh=plsc.VectorSubcoreMesh(**_MESH),
    scratch_types=(
        [pltpu.VMEM_SHARED((_N, 128), jnp.float32)]
        + [pltpu.VMEM((_BLK,), jnp.int32) for _ in range(2 * _Q)]
        + [pltpu.VMEM((_BLK, 128), jnp.float32) for _ in range(_Q)]
        + [pltpu.SemaphoreType.DMA for _ in range(3)]
    ),
)
def _seg(table, src, dst, zeros, out, acc, *bufs):
  c = lax.axis_index("c")
  s = lax.axis_index("s")
  idxs, idxd, rows = bufs[:_Q], bufs[_Q:2 * _Q], bufs[2 * _Q:3 * _Q]
  semi, semg, sems = bufs[3 * _Q:]
  _zero_acc(zeros, acc, s)
  plsc.subcore_barrier()

  w = c * _NS + s
  ngroups = jnp.where(w == 2 * _NS - 1, 27, 26)
  _seg_pipeline(table, src, dst, acc, idxs, idxd, rows, semi, semg, sems,
                w * (78 * _BLK), ngroups)

  @pl.when(w == 2 * _NS - 1)
  def _tail():
    _seg_tail(table, src, dst, acc, idxs[0], idxd[0], rows[0], semg,
              2499 * _BLK)

  plsc.subcore_barrier()
  _write_acc(acc, out, s, c * _N)


@functools.partial(
    pl.kernel,
    out_type=jax.ShapeDtypeStruct((_NC * _N,), jnp.float32),
    mesh=plsc.VectorSubcoreMesh(**_MESH),
    scratch_types=[
        pltpu.VMEM_SHARED((_N,), jnp.float32),
        pltpu.VMEM((_BLK,), jnp.int32),
        pltpu.VMEM((_BLK,), jnp.float32),
        pltpu.VMEM((_N,), jnp.float32),
    ],
)
def _deg_kernel(dst, zeros1, out, deg, idxd, ones, vbuf):
  c = lax.axis_index("c")
  s = lax.axis_index("s")

  @pl.when(s == 0)
  def _zero():
    pltpu.sync_copy(zeros1, deg)

  for j in range(_BLK // 16):
    ones[pl.ds(16 * j, 16)] = jnp.full((16,), 1.0, jnp.float32)
  plsc.subcore_barrier()

  base = c * (_E // _NC) + s * _HALF_STRIDE + jnp.minimum(s, 2) * _BLK
  nblk = jnp.where(s < 2, 79, 78)

  def body(g, carry):
    off = base + g * _BLK
    pltpu.sync_copy(dst.at[pl.ds(off, _BLK)], idxd)
    pltpu.sync_copy(ones, deg.at[idxd], add=True)
    return carry

  lax.fori_loop(0, nblk, body, 0)
  plsc.subcore_barrier()

  @pl.when(s == 0)
  def _writeout():
    pltpu.sync_copy(deg, vbuf)
    pltpu.sync_copy(vbuf, out.at[pl.ds(pl.multiple_of(c * _N, 8), _N)])


_BN = 1000


def _dinv_block(degp_ref):
  deg = degp_ref[0, 0, :] + degp_ref[0, 1, :] + 1.0
  return lax.rsqrt(deg)


def _scale_body(x_ref, degp_ref, xs_ref):
  dinv = _dinv_block(degp_ref)
  xs_ref[...] = x_ref[...] * dinv[:, None]


_scale = pl.pallas_call(
    _scale_body,
    grid=(_N // _BN,),
    in_specs=[
        pl.BlockSpec((_BN, 128), lambda i: (i, 0)),
        pl.BlockSpec((1, 2, _BN), lambda i: (i, 0, 0)),
    ],
    out_specs=pl.BlockSpec((_BN, 128), lambda i: (i, 0)),
    out_shape=jax.ShapeDtypeStruct((_N, 128), jnp.float32),
)


def _mm_body(aggp_ref, xs_ref, degp_ref, w1_ref, b1_ref, w2_ref, y2_ref):
  dinv = _dinv_block(degp_ref)
  z = aggp_ref[0] + aggp_ref[1] + xs_ref[...]
  zw = jnp.dot(z, w1_ref[...], preferred_element_type=jnp.float32)
  h = jnp.maximum(zw * dinv[:, None] + b1_ref[0, :], 0.0)
  y2 = jnp.dot(h, w2_ref[...], preferred_element_type=jnp.float32)
  y2_ref[...] = y2 * dinv[:, None]


_mm = pl.pallas_call(
    _mm_body,
    grid=(_N // _BN,),
    in_specs=[
        pl.BlockSpec((2, _BN, 128), lambda i: (0, i, 0)),
        pl.BlockSpec((_BN, 128), lambda i: (i, 0)),
        pl.BlockSpec((1, 2, _BN), lambda i: (i, 0, 0)),
        pl.BlockSpec((128, 256), lambda i: (0, 0)),
        pl.BlockSpec((1, 256), lambda i: (0, 0)),
        pl.BlockSpec((256, 128), lambda i: (0, 0)),
    ],
    out_specs=pl.BlockSpec((_BN, 128), lambda i: (i, 0)),
    out_shape=jax.ShapeDtypeStruct((_N, 128), jnp.float32),
)


def _fin_body(accp_ref, y2_ref, degp_ref, b2_ref, o_ref):
  dinv = _dinv_block(degp_ref)
  acc = accp_ref[0] + accp_ref[1]
  o_ref[...] = (acc + y2_ref[...]) * dinv[:, None] + b2_ref[0, :]


_fin = pl.pallas_call(
    _fin_body,
    grid=(_N // _BN,),
    in_specs=[
        pl.BlockSpec((2, _BN, 128), lambda i: (0, i, 0)),
        pl.BlockSpec((_BN, 128), lambda i: (i, 0)),
        pl.BlockSpec((1, 2, _BN), lambda i: (i, 0, 0)),
        pl.BlockSpec((1, 128), lambda i: (0, 0)),
    ],
    out_specs=pl.BlockSpec((_BN, 128), lambda i: (i, 0)),
    out_shape=jax.ShapeDtypeStruct((_N, 128), jnp.float32),
)


def kernel(x, edge_index, W1, b1, W2, b2):
  src, dst = edge_index[0], edge_index[1]
  zeros1 = jnp.zeros((_N,), jnp.float32)
  zeros128 = jnp.zeros((_N, 128), jnp.float32)

  degp = _deg_kernel(dst, zeros1).reshape(2, _N)
  degp3 = degp.reshape(2, _N // _BN, _BN).transpose(1, 0, 2)
  xs = _scale(x, degp3)
  aggp = _seg(xs, src, dst, zeros128)
  y2 = _mm(aggp.reshape(2, _N, 128), xs, degp3,
           W1, b1.reshape(1, 256), W2)
  acc2p = _seg(y2, src, dst, zeros128)
  return _fin(acc2p.reshape(2, _N, 128), y2, degp3, b2.reshape(1, 128))


# speedup vs baseline: 27.8465x; 1.0696x over previous
import functools

import jax
import jax.numpy as jnp
from jax import lax
from jax.experimental import pallas as pl
from jax.experimental.pallas import tpu as pltpu
from jax.experimental.pallas import tpu_sc as plsc

_N = 10000
_E = 320000
_NC = 2
_NS = 16
_BLK = 128

_ROWS = 624
_ROWS_LAST = _N - (_NS - 1) * _ROWS

_HALF_STRIDE = 78 * _BLK

_MESH = dict(core_axis_name="c", subcore_axis_name="s", num_cores=_NC,
             num_subcores=_NS)

_Q = 3


def _zero_acc(zeros, acc, s):
  @pl.when(s < _NS - 1)
  def _():
    r0 = pl.multiple_of(s * _ROWS, 8)
    pltpu.sync_copy(zeros.at[pl.ds(r0, _ROWS)], acc.at[pl.ds(r0, _ROWS)])

  @pl.when(s == _NS - 1)
  def _():
    r0 = (_NS - 1) * _ROWS
    pltpu.sync_copy(zeros.at[pl.ds(r0, _ROWS_LAST)],
                    acc.at[pl.ds(r0, _ROWS_LAST)])


def _write_acc(acc, out, s, out_base):
  @pl.when(s < _NS - 1)
  def _():
    r0 = pl.multiple_of(s * _ROWS, 8)
    pltpu.sync_copy(acc.at[pl.ds(r0, _ROWS)],
                    out.at[pl.ds(pl.multiple_of(out_base + r0, 8), _ROWS)])

  @pl.when(s == _NS - 1)
  def _():
    r0 = (_NS - 1) * _ROWS
    pltpu.sync_copy(
        acc.at[pl.ds(r0, _ROWS_LAST)],
        out.at[pl.ds(pl.multiple_of(out_base + r0, 8), _ROWS_LAST)])


def _seg_pipeline(table, srcarr, dstarr, acc, idxs, idxd, rows,
                  semi, semg, sems, eoff, ngroups):
  def body(h, carry):
    o = eoff + h * (_Q * _BLK)

    @pl.when(h >= 1)
    def _drain_prev():
      for j in range(_Q):
        pltpu.make_async_copy(rows[j], acc.at[idxd[j]], sems).wait()

    di = []
    for j in range(_Q):
      di.append(pltpu.async_copy(
          srcarr.at[pl.ds(o + j * _BLK, _BLK)], idxs[j], semi))
      di.append(pltpu.async_copy(
          dstarr.at[pl.ds(o + j * _BLK, _BLK)], idxd[j], semi))
    for d in di:
      d.wait()
    dg = [pltpu.async_copy(table.at[idxs[j]], rows[j], semg)
          for j in range(_Q)]
    for d in dg:
      d.wait()
    for j in range(_Q):
      pltpu.async_copy(rows[j], acc.at[idxd[j]], sems, add=True)
    return carry

  lax.fori_loop(0, ngroups, body, 0)
  for j in range(_Q):
    pltpu.make_async_copy(rows[j], acc.at[idxd[j]], sems).wait()


def _seg_tail(table, srcarr, dstarr, acc, idxs0, idxd0, rows0, semg, eoff):
  pltpu.sync_copy(srcarr.at[pl.ds(eoff, _BLK)], idxs0)
  pltpu.sync_copy(dstarr.at[pl.ds(eoff, _BLK)], idxd0)
  pltpu.async_copy(table.at[idxs0], rows0, semg).wait()
  pltpu.sync_copy(rows0, acc.at[idxd0], add=True)


@functools.partial(
    pl.kernel,
    out_type=jax.ShapeDtypeStruct((_NC * _N, 128), jnp.float32),
    mesh=plsc.VectorSubcoreMesh(**_MESH),
    scratch_types=(
        [pltpu.VMEM_SHARED((_N, 128), jnp.float32)]
        + [pltpu.VMEM((_BLK,), jnp.int32) for _ in range(2 * _Q)]
        + [pltpu.VMEM((_BLK, 128), jnp.float32) for _ in range(_Q)]
        + [pltpu.SemaphoreType.DMA for _ in range(3)]
    ),
)
def _seg(table, src, dst, zeros, out, acc, *bufs):
  c = lax.axis_index("c")
  s = lax.axis_index("s")
  idxs, idxd, rows = bufs[:_Q], bufs[_Q:2 * _Q], bufs[2 * _Q:3 * _Q]
  semi, semg, sems = bufs[3 * _Q:]
  _zero_acc(zeros, acc, s)
  plsc.subcore_barrier()

  w = c * _NS + s
  ngroups = jnp.where(w == 2 * _NS - 1, 27, 26)
  _seg_pipeline(table, src, dst, acc, idxs, idxd, rows, semi, semg, sems,
                w * (78 * _BLK), ngroups)

  @pl.when(w == 2 * _NS - 1)
  def _tail():
    _seg_tail(table, src, dst, acc, idxs[0], idxd[0], rows[0], semg,
              2499 * _BLK)

  plsc.subcore_barrier()
  _write_acc(acc, out, s, c * _N)


@functools.partial(
    pl.kernel,
    out_type=jax.ShapeDtypeStruct((_NC * _N,), jnp.float32),
    mesh=plsc.VectorSubcoreMesh(**_MESH),
    scratch_types=(
        [pltpu.VMEM_SHARED((_N,), jnp.float32)]
        + [pltpu.VMEM((_BLK,), jnp.int32) for _ in range(_Q)]
        + [pltpu.VMEM((_BLK,), jnp.float32),
           pltpu.VMEM((_N,), jnp.float32),
           pltpu.SemaphoreType.DMA,
           pltpu.SemaphoreType.DMA]
    ),
)
def _deg_kernel(dst, zeros1, out, deg, idxd0, idxd1, idxd2, ones, vbuf,
                semi, sems):
  c = lax.axis_index("c")
  s = lax.axis_index("s")
  idxd = (idxd0, idxd1, idxd2)

  @pl.when(s == 0)
  def _zero():
    pltpu.sync_copy(zeros1, deg)

  for j in range(_BLK // 16):
    ones[pl.ds(16 * j, 16)] = jnp.full((16,), 1.0, jnp.float32)
  plsc.subcore_barrier()

  base = c * (_E // _NC) + s * _HALF_STRIDE

  def body(h, carry):
    o = base + h * (_Q * _BLK)

    @pl.when(h >= 1)
    def _drain_prev():
      for j in range(_Q):
        pltpu.make_async_copy(ones, deg.at[idxd[j]], sems).wait()

    di = [pltpu.async_copy(dst.at[pl.ds(o + j * _BLK, _BLK)], idxd[j],
                           semi) for j in range(_Q)]
    for d in di:
      d.wait()
    for j in range(_Q):
      pltpu.async_copy(ones, deg.at[idxd[j]], sems, add=True)
    return carry

  lax.fori_loop(0, 26, body, 0)
  for j in range(_Q):
    pltpu.make_async_copy(ones, deg.at[idxd[j]], sems).wait()

  @pl.when(s == _NS - 1)
  def _tail():
    def tbody(g, carry):
      off = base + 78 * _BLK + g * _BLK
      pltpu.sync_copy(dst.at[pl.ds(off, _BLK)], idxd0)
      pltpu.sync_copy(ones, deg.at[idxd0], add=True)
      return carry
    lax.fori_loop(0, 2, tbody, 0)

  plsc.subcore_barrier()

  @pl.when(s == 0)
  def _writeout():
    pltpu.sync_copy(deg, vbuf)
    pltpu.sync_copy(vbuf, out.at[pl.ds(pl.multiple_of(c * _N, 8), _N)])


_BN = 1000


def _dinv_block(degp_ref):
  deg = degp_ref[0, 0, :] + degp_ref[0, 1, :] + 1.0
  return lax.rsqrt(deg)


def _scale_body(x_ref, degp_ref, xs_ref):
  dinv = _dinv_block(degp_ref)
  xs_ref[...] = x_ref[...] * dinv[:, None]


_scale = pl.pallas_call(
    _scale_body,
    grid=(_N // _BN,),
    in_specs=[
        pl.BlockSpec((_BN, 128), lambda i: (i, 0)),
        pl.BlockSpec((1, 2, _BN), lambda i: (i, 0, 0)),
    ],
    out_specs=pl.BlockSpec((_BN, 128), lambda i: (i, 0)),
    out_shape=jax.ShapeDtypeStruct((_N, 128), jnp.float32),
)


def _mm_body(aggp_ref, xs_ref, degp_ref, w1_ref, b1_ref, w2_ref, y2_ref):
  dinv = _dinv_block(degp_ref)
  z = aggp_ref[0] + aggp_ref[1] + xs_ref[...]
  zw = jnp.dot(z, w1_ref[...], preferred_element_type=jnp.float32)
  h = jnp.maximum(zw * dinv[:, None] + b1_ref[0, :], 0.0)
  y2 = jnp.dot(h, w2_ref[...], preferred_element_type=jnp.float32)
  y2_ref[...] = y2 * dinv[:, None]


_mm = pl.pallas_call(
    _mm_body,
    grid=(_N // _BN,),
    in_specs=[
        pl.BlockSpec((2, _BN, 128), lambda i: (0, i, 0)),
        pl.BlockSpec((_BN, 128), lambda i: (i, 0)),
        pl.BlockSpec((1, 2, _BN), lambda i: (i, 0, 0)),
        pl.BlockSpec((128, 256), lambda i: (0, 0)),
        pl.BlockSpec((1, 256), lambda i: (0, 0)),
        pl.BlockSpec((256, 128), lambda i: (0, 0)),
    ],
    out_specs=pl.BlockSpec((_BN, 128), lambda i: (i, 0)),
    out_shape=jax.ShapeDtypeStruct((_N, 128), jnp.float32),
)


def _fin_body(accp_ref, y2_ref, degp_ref, b2_ref, o_ref):
  dinv = _dinv_block(degp_ref)
  acc = accp_ref[0] + accp_ref[1]
  o_ref[...] = (acc + y2_ref[...]) * dinv[:, None] + b2_ref[0, :]


_fin = pl.pallas_call(
    _fin_body,
    grid=(_N // _BN,),
    in_specs=[
        pl.BlockSpec((2, _BN, 128), lambda i: (0, i, 0)),
        pl.BlockSpec((_BN, 128), lambda i: (i, 0)),
        pl.BlockSpec((1, 2, _BN), lambda i: (i, 0, 0)),
        pl.BlockSpec((1, 128), lambda i: (0, 0)),
    ],
    out_specs=pl.BlockSpec((_BN, 128), lambda i: (i, 0)),
    out_shape=jax.ShapeDtypeStruct((_N, 128), jnp.float32),
)


def kernel(x, edge_index, W1, b1, W2, b2):
  src, dst = edge_index[0], edge_index[1]
  zeros1 = jnp.zeros((_N,), jnp.float32)
  zeros128 = jnp.zeros((_N, 128), jnp.float32)

  degp = _deg_kernel(dst, zeros1).reshape(2, _N)
  degp3 = degp.reshape(2, _N // _BN, _BN).transpose(1, 0, 2)
  xs = _scale(x, degp3)
  aggp = _seg(xs, src, dst, zeros128)
  y2 = _mm(aggp.reshape(2, _N, 128), xs, degp3,
           W1, b1.reshape(1, 256), W2)
  acc2p = _seg(y2, src, dst, zeros128)
  return _fin(acc2p.reshape(2, _N, 128), y2, degp3, b2.reshape(1, 128))
